# baseline (speedup 1.0000x reference)
import functools

import jax
import jax.numpy as jnp
from jax import lax
from jax.experimental import pallas as pl
from jax.experimental.pallas import tpu as pltpu
from jax.experimental.pallas import tpu_sc as plsc

BATCH = 4096
HIST = 50
EMBED = 64
TAGS = 1000

NC, NS, LANES = 2, 16, 16
NW = NC * NS
B_PER_W = BATCH // NW
CHUNK_B = 1
CHUNK_I = CHUNK_B * HIST
N_CHUNKS = B_PER_W // CHUNK_B
NBUF = 16
EV = EMBED // LANES


def _sc_pool_body(x_hbm, table_hbm, out_hbm, idx_v, rows_v, out_v, sems):
    wid = lax.axis_index("s") * NC + lax.axis_index("c")
    pltpu.sync_copy(x_hbm.at[pl.ds(wid * B_PER_W, B_PER_W)], idx_v)

    def start(g, b):
        pltpu.async_copy(table_hbm.at[idx_v.at[g]], rows_v.at[b], sems.at[b])

    for b in range(NBUF):
        start(b, b)

    def outer(t, carry):
        for b in range(NBUF):
            g = t * NBUF + b
            pltpu.make_async_copy(
                table_hbm.at[idx_v.at[g]], rows_v.at[b], sems.at[b]
            ).wait()
            for k in range(EV):
                acc = rows_v[b, 0, pl.ds(k * LANES, LANES)]
                for j in range(1, HIST):
                    acc = acc + rows_v[b, j, pl.ds(k * LANES, LANES)]
                out_v[g, pl.ds(k * LANES, LANES)] = acc
            nxt = g + NBUF

            @pl.when(nxt < N_CHUNKS)
            def _():
                start(nxt, b)

        return carry

    lax.fori_loop(0, N_CHUNKS // NBUF, outer, 0)
    pltpu.sync_copy(out_v, out_hbm.at[wid])


@functools.cache
def _sc_pool():
    return functools.partial(
        pl.kernel,
        out_type=jax.ShapeDtypeStruct((NW, B_PER_W, EMBED), jnp.float32),
        mesh=plsc.VectorSubcoreMesh(core_axis_name="c", subcore_axis_name="s"),
        compiler_params=pltpu.CompilerParams(use_tc_tiling_on_sc=False),
        scratch_types=[
            pltpu.VMEM((B_PER_W, HIST), jnp.int32),
            pltpu.VMEM((NBUF, HIST, EMBED), jnp.float32),
            pltpu.VMEM((B_PER_W, EMBED), jnp.float32),
            pltpu.SemaphoreType.DMA((NBUF,)),
        ],
    )(_sc_pool_body)


BM = 512


def _dense_body(p_ref, w_ref, b_ref, o_ref):
    x = p_ref[...]
    w = w_ref[...]
    s = lax.dot_general(
        x, w, (((1,), (1,)), ((), ())), preferred_element_type=jnp.float32
    )
    s = s + b_ref[...]
    m = jnp.max(s, axis=-1, keepdims=True)
    e = jnp.exp(s - m)
    lse = jnp.log(jnp.sum(e, axis=-1, keepdims=True)) + m
    o_ref[...] = s - lse


_dense = pl.pallas_call(
    _dense_body,
    grid=(BATCH // BM,),
    in_specs=[
        pl.BlockSpec((BM, EMBED), lambda i: (i, 0)),
        pl.BlockSpec((TAGS, EMBED), lambda i: (0, 0)),
        pl.BlockSpec((1, TAGS), lambda i: (0, 0)),
    ],
    out_specs=pl.BlockSpec((BM, TAGS), lambda i: (i, 0)),
    out_shape=jax.ShapeDtypeStruct((BATCH, TAGS), jnp.float32),
    compiler_params=pltpu.CompilerParams(dimension_semantics=("parallel",)),
)


def kernel(x, embed_table, W_lin, bow_bias):
    x32 = x.astype(jnp.int32)
    pooled = _sc_pool()(x32, embed_table)
    pooled = pooled.reshape(BATCH, EMBED)
    return _dense(pooled, W_lin, bow_bias.reshape(1, TAGS))


# speedup vs baseline: 1.4817x; 1.4817x over previous
import functools

import jax
import jax.numpy as jnp
from jax import lax
from jax.experimental import pallas as pl
from jax.experimental.pallas import tpu as pltpu
from jax.experimental.pallas import tpu_sc as plsc

BATCH = 4096
HIST = 50
EMBED = 64
TAGS = 1000

NC, NS, LANES = 2, 16, 16
NW = NC * NS
B_PER_W = BATCH // NW
NBUF = 5
EV = EMBED // LANES
OUT_ROWS = B_PER_W * EMBED // 128


def _sc_pool_body(xt_hbm, table_hbm, out_hbm, idx_v, rows_v, out_v, sems):
    wid = lax.axis_index("s") * NC + lax.axis_index("c")
    pltpu.sync_copy(xt_hbm.at[pl.ds(wid * HIST, HIST)], idx_v)

    zeros = jnp.zeros((LANES,), jnp.float32)
    for r in range(OUT_ROWS):
        for k in range(128 // LANES):
            out_v[r, pl.ds(k * LANES, LANES)] = zeros

    def start(g, b):
        pltpu.async_copy(table_hbm.at[idx_v.at[g]], rows_v.at[b], sems.at[b])

    for b in range(NBUF):
        start(b, b)

    def outer(t, carry):
        for b in range(NBUF):
            g = t * NBUF + b
            pltpu.make_async_copy(
                table_hbm.at[idx_v.at[g]], rows_v.at[b], sems.at[b]
            ).wait()
            @plsc.parallel_loop(0, OUT_ROWS, unroll=8)
            def _(rp):
                for half in range(2):
                    for k in range(EV):
                        plsc.addupdate(
                            out_v.at[rp, pl.ds(half * EMBED + k * LANES, LANES)],
                            rows_v[b, 2 * rp + half, pl.ds(k * LANES, LANES)],
                        )
            nxt = g + NBUF

            @pl.when(nxt < HIST)
            def _():
                start(nxt, b)

        return carry

    lax.fori_loop(0, HIST // NBUF, outer, 0)
    pltpu.sync_copy(out_v, out_hbm.at[pl.ds(wid * OUT_ROWS, OUT_ROWS)])


@functools.cache
def _sc_pool():
    return functools.partial(
        pl.kernel,
        out_type=jax.ShapeDtypeStruct((NW * OUT_ROWS, 128), jnp.float32),
        mesh=plsc.VectorSubcoreMesh(core_axis_name="c", subcore_axis_name="s"),
        compiler_params=pltpu.CompilerParams(use_tc_tiling_on_sc=False),
        scratch_types=[
            pltpu.VMEM((HIST, B_PER_W), jnp.int32),
            pltpu.VMEM((NBUF, B_PER_W, EMBED), jnp.float32),
            pltpu.VMEM((OUT_ROWS, 128), jnp.float32),
            pltpu.SemaphoreType.DMA((NBUF,)),
        ],
    )(_sc_pool_body)


BM = 512


def _dense_body(p_ref, w_ref, b_ref, o_ref):
    x = p_ref[...]
    w = w_ref[...]
    s = lax.dot_general(
        x, w, (((1,), (1,)), ((), ())), preferred_element_type=jnp.float32
    )
    s = s + b_ref[...]
    m = jnp.max(s, axis=-1, keepdims=True)
    e = jnp.exp(s - m)
    lse = jnp.log(jnp.sum(e, axis=-1, keepdims=True)) + m
    o_ref[...] = s - lse


_dense = pl.pallas_call(
    _dense_body,
    grid=(BATCH // BM,),
    in_specs=[
        pl.BlockSpec((BM, EMBED), lambda i: (i, 0)),
        pl.BlockSpec((TAGS, EMBED), lambda i: (0, 0)),
        pl.BlockSpec((1, TAGS), lambda i: (0, 0)),
    ],
    out_specs=pl.BlockSpec((BM, TAGS), lambda i: (i, 0)),
    out_shape=jax.ShapeDtypeStruct((BATCH, TAGS), jnp.float32),
    compiler_params=pltpu.CompilerParams(dimension_semantics=("parallel",)),
)


def kernel(x, embed_table, W_lin, bow_bias):
    xt = (
        x.astype(jnp.int32)
        .reshape(NW, B_PER_W, HIST)
        .transpose(0, 2, 1)
        .reshape(NW * HIST, B_PER_W)
    )
    pooled = _sc_pool()(xt, embed_table)
    pooled = pooled.reshape(BATCH, EMBED)
    return _dense(pooled, W_lin, bow_bias.reshape(1, TAGS))


# speedup vs baseline: 1.9159x; 1.2930x over previous
import functools

import jax
import jax.numpy as jnp
from jax import lax
from jax.experimental import pallas as pl
from jax.experimental.pallas import tpu as pltpu
from jax.experimental.pallas import tpu_sc as plsc

BATCH = 4096
HIST = 50
EMBED = 64
TAGS = 1000
VOCAB = 100000

NC, NS, LANES = 2, 16, 16
NW = NC * NS
B_PER_W = BATCH // NW
NBUF = 5
EV = EMBED // LANES
OUT_ROWS = B_PER_W * EMBED // 128


BN = 4096


def _prep_body(t_ref, o_ref):
    blk = t_ref[...]
    tr = jnp.transpose(blk, (1, 0))
    t3 = tr.reshape(BN // 2, 2, EMBED)
    o_ref[:, :EMBED] = t3[:, 0, :]
    o_ref[:, EMBED:] = t3[:, 1, :]


_prep = pl.pallas_call(
    _prep_body,
    grid=(pl.cdiv(VOCAB, BN),),
    in_specs=[pl.BlockSpec((EMBED, BN), lambda i: (0, i))],
    out_specs=pl.BlockSpec((BN // 2, 128), lambda i: (i, 0)),
    out_shape=jax.ShapeDtypeStruct((VOCAB // 2, 128), jnp.float32),
    compiler_params=pltpu.CompilerParams(dimension_semantics=("parallel",)),
)


def _sc_pool_body(xt_hbm, table_hbm, out_hbm, idx_v, rows_v, out_v, sems):
    wid = lax.axis_index("s") * NC + lax.axis_index("c")
    pltpu.sync_copy(xt_hbm.at[pl.ds(wid * HIST, HIST)], idx_v)

    zeros = jnp.zeros((LANES,), jnp.float32)
    for r in range(OUT_ROWS):
        for k in range(128 // LANES):
            out_v[r, pl.ds(k * LANES, LANES)] = zeros

    def start(g, b):
        pltpu.async_copy(table_hbm.at[idx_v.at[g]], rows_v.at[b], sems.at[b])

    for b in range(NBUF):
        start(b, b)

    def outer(t, carry):
        for b in range(NBUF):
            g = t * NBUF + b
            pltpu.make_async_copy(
                table_hbm.at[idx_v.at[g]], rows_v.at[b], sems.at[b]
            ).wait()

            @plsc.parallel_loop(0, OUT_ROWS, unroll=8)
            def _(rp):
                for half in range(2):
                    for k in range(EV):
                        plsc.addupdate(
                            out_v.at[rp, pl.ds(half * EMBED + k * LANES, LANES)],
                            rows_v[b, 2 * rp + half, pl.ds(k * LANES, LANES)],
                        )

            nxt = g + NBUF

            @pl.when(nxt < HIST)
            def _():
                start(nxt, b)

        return carry

    lax.fori_loop(0, HIST // NBUF, outer, 0)
    pltpu.sync_copy(out_v, out_hbm.at[pl.ds(wid * OUT_ROWS, OUT_ROWS)])


@functools.cache
def _sc_pool():
    return functools.partial(
        pl.kernel,
        out_type=jax.ShapeDtypeStruct((NW * OUT_ROWS, 128), jnp.float32),
        mesh=plsc.VectorSubcoreMesh(core_axis_name="c", subcore_axis_name="s"),
        compiler_params=pltpu.CompilerParams(use_tc_tiling_on_sc=False),
        scratch_types=[
            pltpu.VMEM((HIST, B_PER_W), jnp.int32),
            pltpu.VMEM((NBUF, B_PER_W, EMBED), jnp.float32),
            pltpu.VMEM((OUT_ROWS, 128), jnp.float32),
            pltpu.SemaphoreType.DMA((NBUF,)),
        ],
    )(_sc_pool_body)


BM = 512


def _dense_body(p_ref, w_ref, b_ref, o_ref):
    x = p_ref[...]
    w = w_ref[...]
    s = lax.dot_general(
        w, x, (((0,), (1,)), ((), ())), preferred_element_type=jnp.float32
    )
    s = s + b_ref[...]
    m = jnp.max(s, axis=0, keepdims=True)
    e = jnp.exp(s - m)
    lse = jnp.log(jnp.sum(e, axis=0, keepdims=True)) + m
    o_ref[...] = s - lse


_dense = pl.pallas_call(
    _dense_body,
    grid=(BATCH // BM,),
    in_specs=[
        pl.BlockSpec((BM, EMBED), lambda i: (i, 0)),
        pl.BlockSpec((EMBED, TAGS), lambda i: (0, 0)),
        pl.BlockSpec((TAGS, 1), lambda i: (0, 0)),
    ],
    out_specs=pl.BlockSpec((TAGS, BM), lambda i: (0, i)),
    out_shape=jax.ShapeDtypeStruct((TAGS, BATCH), jnp.float32),
    compiler_params=pltpu.CompilerParams(dimension_semantics=("parallel",)),
)


def kernel(x, embed_table, W_lin, bow_bias):
    table_lin = _prep(embed_table.T)

    xt = (
        x.astype(jnp.int32)
        .reshape(NW, B_PER_W, HIST)
        .transpose(0, 2, 1)
        .reshape(NW * HIST, B_PER_W)
    )
    pooled = _sc_pool()(xt, table_lin.reshape(VOCAB, EMBED))
    pooled = pooled.reshape(BATCH, EMBED)

    out_t = _dense(pooled, W_lin.T, bow_bias.reshape(TAGS, 1))
    return out_t.T


# speedup vs baseline: 2.0496x; 1.0698x over previous
import functools

import jax
import jax.numpy as jnp
from jax import lax
from jax.experimental import pallas as pl
from jax.experimental.pallas import tpu as pltpu
from jax.experimental.pallas import tpu_sc as plsc

BATCH = 4096
HIST = 50
EMBED = 64
TAGS = 1000
VOCAB = 100000

NC, NS, LANES = 2, 16, 16
NW = NC * NS
B_PER_W = BATCH // NW
NBUF = 5
EV = EMBED // LANES
OUT_ROWS = B_PER_W * EMBED // 128


BN = 4096


NBLK = pl.cdiv(VOCAB, BN)
VPAD = NBLK * BN


def _prep_body(t_ref, eye_ref, o_ref):
    tr = lax.dot_general(
        t_ref[...], eye_ref[...], (((0,), (0,)), ((), ())),
        preferred_element_type=jnp.float32,
    )
    o_ref[...] = jnp.concatenate([tr[: BN // 2], tr[BN // 2 :]], axis=1)


_prep = pl.pallas_call(
    _prep_body,
    grid=(NBLK,),
    in_specs=[
        pl.BlockSpec((EMBED, BN), lambda i: (0, i)),
        pl.BlockSpec((EMBED, EMBED), lambda i: (0, 0)),
    ],
    out_specs=pl.BlockSpec((BN // 2, 128), lambda i: (i, 0)),
    out_shape=jax.ShapeDtypeStruct((VPAD // 2, 128), jnp.float32),
    compiler_params=pltpu.CompilerParams(dimension_semantics=("parallel",)),
)


def _sc_pool_body(xt_hbm, table_hbm, out_hbm, idx_v, rows_v, out_v, sems):
    wid = lax.axis_index("s") * NC + lax.axis_index("c")
    pltpu.sync_copy(xt_hbm.at[pl.ds(wid * HIST, HIST)], idx_v)

    def remap(g, carry):
        for q in range(B_PER_W // LANES):
            v = idx_v[g, pl.ds(q * LANES, LANES)]
            r = v & (BN - 1)
            idx_v[g, pl.ds(q * LANES, LANES)] = (
                (v - r) + ((r & (BN // 2 - 1)) << 1) + (r >> 11)
            )
        return carry

    lax.fori_loop(0, HIST, remap, 0)

    zeros = jnp.zeros((LANES,), jnp.float32)
    for r in range(OUT_ROWS):
        for k in range(128 // LANES):
            out_v[r, pl.ds(k * LANES, LANES)] = zeros

    def start(g, b):
        pltpu.async_copy(table_hbm.at[idx_v.at[g]], rows_v.at[b], sems.at[b])

    for b in range(NBUF):
        start(b, b)

    def outer(t, carry):
        for b in range(NBUF):
            g = t * NBUF + b
            pltpu.make_async_copy(
                table_hbm.at[idx_v.at[g]], rows_v.at[b], sems.at[b]
            ).wait()

            @plsc.parallel_loop(0, OUT_ROWS, unroll=8)
            def _(rp):
                vals = [
                    rows_v[b, 2 * rp + half, pl.ds(k * LANES, LANES)]
                    for half in range(2)
                    for k in range(EV)
                ]
                for i, val in enumerate(vals):
                    plsc.addupdate(
                        out_v.at[rp, pl.ds(i * LANES, LANES)], val
                    )

            nxt = g + NBUF

            @pl.when(nxt < HIST)
            def _():
                start(nxt, b)

        return carry

    lax.fori_loop(0, HIST // NBUF, outer, 0)
    pltpu.sync_copy(out_v, out_hbm.at[pl.ds(wid * OUT_ROWS, OUT_ROWS)])


@functools.cache
def _sc_pool():
    return functools.partial(
        pl.kernel,
        out_type=jax.ShapeDtypeStruct((NW * OUT_ROWS, 128), jnp.float32),
        mesh=plsc.VectorSubcoreMesh(core_axis_name="c", subcore_axis_name="s"),
        compiler_params=pltpu.CompilerParams(use_tc_tiling_on_sc=False),
        scratch_types=[
            pltpu.VMEM((HIST, B_PER_W), jnp.int32),
            pltpu.VMEM((NBUF, B_PER_W, EMBED), jnp.float32),
            pltpu.VMEM((OUT_ROWS, 128), jnp.float32),
            pltpu.SemaphoreType.DMA((NBUF,)),
        ],
    )(_sc_pool_body)


BM = 512


def _dense_body(p_ref, w_ref, b_ref, o_ref):
    x = p_ref[...]
    w = w_ref[...]
    s = lax.dot_general(
        w, x, (((0,), (1,)), ((), ())), preferred_element_type=jnp.float32
    )
    s = s + b_ref[...]
    m = jnp.max(s, axis=0, keepdims=True)
    e = jnp.exp(s - m)
    lse = jnp.log(jnp.sum(e, axis=0, keepdims=True)) + m
    o_ref[...] = s - lse


_dense = pl.pallas_call(
    _dense_body,
    grid=(BATCH // BM,),
    in_specs=[
        pl.BlockSpec((BM, EMBED), lambda i: (i, 0)),
        pl.BlockSpec((EMBED, TAGS), lambda i: (0, 0)),
        pl.BlockSpec((TAGS, 1), lambda i: (0, 0)),
    ],
    out_specs=pl.BlockSpec((TAGS, BM), lambda i: (0, i)),
    out_shape=jax.ShapeDtypeStruct((TAGS, BATCH), jnp.float32),
    compiler_params=pltpu.CompilerParams(dimension_semantics=("parallel",)),
)


def kernel(x, embed_table, W_lin, bow_bias):
    eye = jnp.eye(EMBED, dtype=jnp.float32)
    table_lin = _prep(embed_table.T, eye)

    xt = (
        x.astype(jnp.int32)
        .reshape(NW, B_PER_W, HIST)
        .transpose(0, 2, 1)
        .reshape(NW * HIST, B_PER_W)
    )
    pooled = _sc_pool()(xt, table_lin.reshape(VPAD, EMBED))
    pooled = pooled.reshape(BATCH, EMBED)

    out_t = _dense(pooled, W_lin.T, bow_bias.reshape(TAGS, 1))
    return out_t.T
